# baseline (speedup 1.0000x reference)
import functools

import jax
import jax.numpy as jnp
from jax import lax
from jax.experimental import pallas as pl
from jax.experimental.pallas import tpu as pltpu
from jax.experimental.pallas import tpu_sc as plsc

F32 = jnp.float32

_PH = {0: (1, -1), 1: (0, 0), 2: (1, 0), 3: (0, 1)}

_CT_TAPS = {0: ((1, 1), (3, 0)), 1: ((0, 2), (2, 1))}

_GATHER_DNUMS = lax.GatherDimensionNumbers(
    offset_dims=(), collapsed_slice_dims=(0,), start_index_map=(0,))


def _lane_shuffle(x, perm):
    return lax.gather(x, perm[:, None], _GATHER_DNUMS, (1,),
                      mode=lax.GatherScatterMode.PROMISE_IN_BOUNDS)


def _mm(a, b):
    return jnp.dot(a, b, preferred_element_type=F32)


def _pad_hw(t):
    B, H, W, C = t.shape
    t = jnp.concatenate(
        [jnp.zeros((B, 1, W, C), F32), t, jnp.zeros((B, 1, W, C), F32)], axis=1)
    t = jnp.concatenate(
        [jnp.zeros((B, H + 2, 1, C), F32), t, jnp.zeros((B, H + 2, 1, C), F32)],
        axis=2)
    return t


def _conv3x3(tp, w, b):
    B, Hp, Wp, C = tp.shape
    H, W = Hp - 2, Wp - 2
    Co = w.shape[-1]
    acc = jnp.zeros((B * H * W, Co), F32)
    for dy in (-1, 0, 1):
        for dx in (-1, 0, 1):
            sl = tp[:, 1 + dy:1 + dy + H, 1 + dx:1 + dx + W, :]
            acc = acc + _mm(sl.reshape(B * H * W, C), w[dy + 1, dx + 1])
    return (acc + b[None, :]).reshape(B, H, W, Co)


def _res_block(h, wa, ba, wb, bb):
    B, H, W, C = h.shape
    t = jnp.maximum(h, 0.0)
    t = _conv3x3(_pad_hw(t), wa, ba)
    t = jnp.maximum(t, 0.0)
    t2 = _mm(t.reshape(B * H * W, C), wb) + bb[None, :]
    return h + t2.reshape(B, H, W, C)


def _k1_body(xpp_ref, w_ref, b_ref, out_ref):
    xpp = xpp_ref[...]
    acc = jnp.zeros((8 * 32 * 32, 128), F32)
    for dy in (-1, 0, 1):
        for dx in (-1, 0, 1):
            sl = xpp[:, 1 + dy:33 + dy, 1 + dx:33 + dx, :]
            acc = acc + _mm(sl.reshape(8192, 12), w_ref[dy + 1, dx + 1])
    out_ref[...] = (acc + b_ref[...][None, :]).reshape(8, 32, 32, 128)


def _k2_body(h1p_ref, w2_ref, b2_ref, wa1_ref, ba1_ref, wb1_ref, bb1_ref,
             wa2_ref, ba2_ref, wb2_ref, bb2_ref, wout_ref, bout_ref, out_ref):
    h1p = h1p_ref[...]
    acc = jnp.zeros((8 * 16 * 16, 128), F32)
    for ky in range(4):
        py, dy = _PH[ky]
        for kx in range(4):
            px, dx = _PH[kx]
            ph = py * 2 + px
            sl = h1p[:, 1 + dy:17 + dy, 1 + dx:17 + dx,
                     ph * 128:(ph + 1) * 128]
            acc = acc + _mm(sl.reshape(2048, 128), w2_ref[ky, kx])
    h = (acc + b2_ref[...][None, :]).reshape(8, 16, 16, 128)
    h = _res_block(h, wa1_ref[...], ba1_ref[...], wb1_ref[...], bb1_ref[...])
    h = _res_block(h, wa2_ref[...], ba2_ref[...], wb2_ref[...], bb2_ref[...])
    z = _mm(h.reshape(2048, 128), wout_ref[...]) + bout_ref[...][None, :]
    out_ref[...] = z.reshape(8, 16, 16, 16)


def _k3_body(z_ref, cb_ref, dist_ref):
    cb = cb_ref[...]
    dots = lax.dot_general(z_ref[...], cb, (((1,), (1,)), ((), ())),
                           preferred_element_type=F32)
    cbsq = jnp.sum(cb * cb, axis=1)
    dist_ref[...] = cbsq[None, :] - 2.0 * dots


def _k4_body(lat_ref, woct_ref, boct_ref, wa1_ref, ba1_ref, wb1_ref, bb1_ref,
             wa2_ref, ba2_ref, wb2_ref, bb2_ref, wct_ref, bct_ref, out_ref):
    lat = lat_ref[...]
    h = _mm(lat.reshape(2048, 16), woct_ref[...]) + boct_ref[...][None, :]
    h = h.reshape(8, 16, 16, 128)
    h = _res_block(h, wa1_ref[...], ba1_ref[...], wb1_ref[...], bb1_ref[...])
    h = _res_block(h, wa2_ref[...], ba2_ref[...], wb2_ref[...], bb2_ref[...])
    hp = _pad_hw(h)
    outs = []
    for a in (0, 1):
        for b in (0, 1):
            acc = jnp.zeros((2048, 128), F32)
            for ky, oy in _CT_TAPS[a]:
                for kx, ox in _CT_TAPS[b]:
                    sl = hp[:, oy:oy + 16, ox:ox + 16, :]
                    acc = acc + _mm(sl.reshape(2048, 128), wct_ref[ky, kx])
            acc = acc + bct_ref[...][None, :]
            outs.append(acc.reshape(8, 16, 16, 128))
    out_ref[...] = jnp.concatenate(outs, axis=3)


def _k5_body(rp_ref, wct_ref, bct_ref, out_ref):
    rp = rp_ref[...]
    outs = []
    for a in (0, 1):
        for b in (0, 1):
            acc = jnp.zeros((8192, 3), F32)
            for ky, oy in _CT_TAPS[a]:
                for kx, ox in _CT_TAPS[b]:
                    sl = rp[:, oy:oy + 32, ox:ox + 32, :]
                    acc = acc + _mm(sl.reshape(8192, 128), wct_ref[ky, kx])
            acc = acc + bct_ref[...][None, :]
            outs.append(acc.reshape(8, 32, 32, 3))
    out_ref[...] = jnp.concatenate(outs, axis=3)


def _vq_sc_body(dist_hbm, cb_hbm, out_hbm, dist_v, idx_v, rows_v, sem):
    wid = lax.axis_index("s") * 2 + lax.axis_index("c")

    @pl.when(wid == 0)
    def _():
        pltpu.sync_copy(dist_hbm, dist_v)
        lanes = lax.broadcasted_iota(jnp.int32, (16,), 0)
        idxs = jnp.zeros((16,), jnp.int32)
        for b in range(8):
            minv = dist_v[b, pl.ds(0, 16)]
            mini = lanes
            for c in range(1, 32):
                v = dist_v[b, pl.ds(c * 16, 16)]
                ii = lanes + c * 16
                m = v < minv
                minv = jnp.where(m, v, minv)
                mini = jnp.where(m, ii, mini)
            for shift in (8, 4, 2, 1):
                perm = jnp.bitwise_xor(lanes, shift)
                ov = _lane_shuffle(minv, perm)
                oi = _lane_shuffle(mini, perm)
                better = (ov < minv) | ((ov == minv) & (oi < mini))
                minv = jnp.where(better, ov, minv)
                mini = jnp.where(better, oi, mini)
            idxs = jnp.where(lanes == b, mini, idxs)
        idx_v[...] = idxs
        pltpu.async_copy(cb_hbm.at[idx_v], rows_v, sem).wait()
        pltpu.sync_copy(rows_v.at[pl.ds(0, 8)], out_hbm)


def _vq_lookup(dist, codebook):
    mesh = plsc.VectorSubcoreMesh(core_axis_name="c", subcore_axis_name="s")
    fn = functools.partial(
        pl.kernel,
        mesh=mesh,
        out_type=jax.ShapeDtypeStruct((8, 4096), F32),
        scratch_types=[
            pltpu.VMEM((8, 512), F32),
            pltpu.VMEM((16,), jnp.int32),
            pltpu.VMEM((16, 4096), F32),
            pltpu.SemaphoreType.DMA,
        ],
    )(_vq_sc_body)
    return fn(dist, codebook)


def _interleave2(o, c):
    B, H, W, _ = o.shape
    o = o.reshape(B, H, W, 2, 2, c).transpose(0, 1, 3, 2, 4, 5)
    return o.reshape(B, 2 * H, 2 * W, c)


def kernel(x, codebook, enc_c1_w, enc_c1_b, enc_c2_w, enc_c2_b, enc_b1a_w,
           enc_b1a_b, enc_b1b_w, enc_b1b_b, enc_b2a_w, enc_b2a_b, enc_b2b_w,
           enc_b2b_b, enc_out_w, enc_out_b, dec_oct_w, dec_oct_b, dec_b1a_w,
           dec_b1a_b, dec_b1b_w, dec_b1b_b, dec_b2a_w, dec_b2a_b, dec_b2b_w,
           dec_b2b_b, dec_ct1_w, dec_ct1_b, dec_ct2_w, dec_ct2_b):
    f32 = F32

    x_nhwc = x.transpose(0, 2, 3, 1)
    xph = x_nhwc.reshape(8, 32, 2, 32, 2, 3).transpose(0, 1, 3, 2, 4, 5)
    xpp = jnp.pad(xph.reshape(8, 32, 32, 12),
                  ((0, 0), (1, 1), (1, 1), (0, 0)))

    w1s = jnp.zeros((3, 3, 12, 128), f32)
    for ky in range(4):
        py, dy = _PH[ky]
        for kx in range(4):
            px, dx = _PH[kx]
            base = (py * 2 + px) * 3
            w1s = w1s.at[dy + 1, dx + 1, base:base + 3, :].set(
                enc_c1_w[:, :, ky, kx].T)

    w2 = enc_c2_w.transpose(2, 3, 1, 0)
    ea1 = enc_b1a_w.transpose(2, 3, 1, 0)
    eb1 = enc_b1b_w[:, :, 0, 0].T
    ea2 = enc_b2a_w.transpose(2, 3, 1, 0)
    eb2 = enc_b2b_w[:, :, 0, 0].T
    wout = enc_out_w[:, :, 0, 0].T
    woct = dec_oct_w[:, :, 0, 0]
    da1 = dec_b1a_w.transpose(2, 3, 1, 0)
    db1 = dec_b1b_w[:, :, 0, 0].T
    da2 = dec_b2a_w.transpose(2, 3, 1, 0)
    db2 = dec_b2b_w[:, :, 0, 0].T
    wct1 = dec_ct1_w.transpose(2, 3, 0, 1)
    wct2 = dec_ct2_w.transpose(2, 3, 0, 1)

    h1 = pl.pallas_call(
        _k1_body,
        out_shape=jax.ShapeDtypeStruct((8, 32, 32, 128), f32),
    )(xpp, w1s, enc_c1_b)

    h1ph = h1.reshape(8, 16, 2, 16, 2, 128).transpose(0, 1, 3, 2, 4, 5)
    h1p = jnp.pad(h1ph.reshape(8, 16, 16, 512),
                  ((0, 0), (1, 1), (1, 1), (0, 0)))
    z_map = pl.pallas_call(
        _k2_body,
        out_shape=jax.ShapeDtypeStruct((8, 16, 16, 16), f32),
    )(h1p, w2, enc_c2_b, ea1, enc_b1a_b, eb1, enc_b1b_b, ea2, enc_b2a_b, eb2,
      enc_b2b_b, wout, enc_out_b)

    z_nchw = z_map.transpose(0, 3, 1, 2)
    z_flat = z_nchw.reshape(8, 4096)
    z_e = z_nchw.reshape(8, 1, 4096)

    dist = pl.pallas_call(
        _k3_body,
        out_shape=jax.ShapeDtypeStruct((8, 512), f32),
    )(z_flat, codebook)
    z_q = _vq_lookup(dist, codebook)

    latent = z_q.reshape(8, 16, 16, 16)

    o4 = pl.pallas_call(
        _k4_body,
        out_shape=jax.ShapeDtypeStruct((8, 16, 16, 512), f32),
    )(latent, woct, dec_oct_b, da1, dec_b1a_b, db1, dec_b1b_b, da2, dec_b2a_b,
      db2, dec_b2b_b, wct1, dec_ct1_b)
    r1 = _interleave2(o4, 128)

    rp = jnp.pad(r1, ((0, 0), (1, 1), (1, 1), (0, 0)))
    o5 = pl.pallas_call(
        _k5_body,
        out_shape=jax.ShapeDtypeStruct((8, 32, 32, 12), f32),
    )(rp, wct2, dec_ct2_b)
    r = _interleave2(o5, 3).transpose(0, 3, 1, 2)

    return (r, z_e, z_q, z_e, z_q)


# speedup vs baseline: 1.4503x; 1.4503x over previous
import functools

import jax
import jax.numpy as jnp
from jax import lax
from jax.experimental import pallas as pl
from jax.experimental.pallas import tpu as pltpu
from jax.experimental.pallas import tpu_sc as plsc

F32 = jnp.float32

_PH = {0: (1, -1), 1: (0, 0), 2: (1, 0), 3: (0, 1)}

_DYS = {0: (-1, 0), 1: (0, 1)}

_CT_TAPS = {0: ((1, 1), (3, 0)), 1: ((0, 2), (2, 1))}

_CT2_ROWSEL = {
    (0, 0): (1, 2, 3, None),
    (1, 0): (None, 0, 1, 2),
    (1, -1): (3, None, None, None),
    (0, 1): (None, None, None, 0),
}
_CT2_SLICES = ((0, 0), (1, 0), (1, -1), (0, 1))

_GATHER_DNUMS = lax.GatherDimensionNumbers(
    offset_dims=(), collapsed_slice_dims=(0,), start_index_map=(0,))


def _lane_shuffle(x, perm):
    return lax.gather(x, perm[:, None], _GATHER_DNUMS, (1,),
                      mode=lax.GatherScatterMode.PROMISE_IN_BOUNDS)


def _mm(a, b):
    return jnp.dot(a, b, preferred_element_type=F32)


def _pad_hw(t):
    B, H, W, C = t.shape
    t = jnp.concatenate(
        [jnp.zeros((B, 1, W, C), F32), t, jnp.zeros((B, 1, W, C), F32)], axis=1)
    t = jnp.concatenate(
        [jnp.zeros((B, H + 2, 1, C), F32), t, jnp.zeros((B, H + 2, 1, C), F32)],
        axis=2)
    return t


def _conv3x3(tp, w, b):
    B, Hp, Wp, C = tp.shape
    H, W = Hp - 2, Wp - 2
    Co = w.shape[-1]
    acc = jnp.zeros((B * H * W, Co), F32)
    for dy in (-1, 0, 1):
        for dx in (-1, 0, 1):
            sl = tp[:, 1 + dy:1 + dy + H, 1 + dx:1 + dx + W, :]
            acc = acc + _mm(sl.reshape(B * H * W, C), w[dy + 1, dx + 1])
    return (acc + b[None, :]).reshape(B, H, W, Co)


def _res_block(h, wa, ba, wb, bb):
    B, H, W, C = h.shape
    t = jnp.maximum(h, 0.0)
    t = _conv3x3(_pad_hw(t), wa, ba)
    t = jnp.maximum(t, 0.0)
    t2 = _mm(t.reshape(B * H * W, C), wb) + bb[None, :]
    return h + t2.reshape(B, H, W, C)


def _enc_body(xq_ref, w1_ref, b1_ref, w2_ref, b2_ref, wa1_ref, ba1_ref,
              wb1_ref, bb1_ref, wa2_ref, ba2_ref, wb2_ref, bb2_ref, wout_ref,
              bout_ref, out_ref):
    xq = xq_ref[...]
    b1 = b1_ref[...]
    h1p = {}
    for py in (0, 1):
        for px in (0, 1):
            acc = jnp.zeros((2048, 128), F32)
            for i, dy in enumerate(_DYS[py]):
                for j, dx in enumerate(_DYS[px]):
                    sl = xq[:, 1 + dy:17 + dy, 1 + dx:17 + dx, :]
                    acc = acc + _mm(sl.reshape(2048, 48), w1_ref[py, px, i, j])
            h1p[(py, px)] = _pad_hw(
                (acc + b1[None, :]).reshape(8, 16, 16, 128))
    acc = jnp.zeros((2048, 128), F32)
    for ky in range(4):
        py, dy = _PH[ky]
        for kx in range(4):
            px, dx = _PH[kx]
            sl = h1p[(py, px)][:, 1 + dy:17 + dy, 1 + dx:17 + dx, :]
            acc = acc + _mm(sl.reshape(2048, 128), w2_ref[ky, kx])
    h = (acc + b2_ref[...][None, :]).reshape(8, 16, 16, 128)
    h = _res_block(h, wa1_ref[...], ba1_ref[...], wb1_ref[...], bb1_ref[...])
    h = _res_block(h, wa2_ref[...], ba2_ref[...], wb2_ref[...], bb2_ref[...])
    z = _mm(h.reshape(2048, 128), wout_ref[...]) + bout_ref[...][None, :]
    out_ref[...] = z.reshape(8, 16, 16, 16)


def _dist_body(z_ref, cb_ref, dist_ref):
    cb = cb_ref[...]
    dots = lax.dot_general(z_ref[...], cb, (((1,), (1,)), ((), ())),
                           preferred_element_type=F32)
    cbsq = jnp.sum(cb * cb, axis=1)
    dist_ref[...] = cbsq[None, :] - 2.0 * dots


def _dec_body(lat_ref, woct_ref, boct_ref, wa1_ref, ba1_ref, wb1_ref, bb1_ref,
              wa2_ref, ba2_ref, wb2_ref, bb2_ref, wct1_ref, bct1_ref,
              wct2_ref, b5_ref, out_ref):
    lat = lat_ref[...]
    h = _mm(lat.reshape(2048, 16), woct_ref[...]) + boct_ref[...][None, :]
    h = h.reshape(8, 16, 16, 128)
    h = _res_block(h, wa1_ref[...], ba1_ref[...], wb1_ref[...], bb1_ref[...])
    h = _res_block(h, wa2_ref[...], ba2_ref[...], wb2_ref[...], bb2_ref[...])
    hp = _pad_hw(h)
    bct1 = bct1_ref[...]
    pp = {}
    for a in (0, 1):
        for b in (0, 1):
            acc = jnp.zeros((2048, 128), F32)
            for ky, oy in _CT_TAPS[a]:
                for kx, ox in _CT_TAPS[b]:
                    sl = hp[:, oy:oy + 16, ox:ox + 16, :]
                    acc = acc + _mm(sl.reshape(2048, 128), wct1_ref[ky, kx])
            pp[(a, b)] = _pad_hw(
                (acc + bct1[None, :]).reshape(8, 16, 16, 128))
    acc = jnp.zeros((2048, 48), F32)
    for si, (pa, sy) in enumerate(_CT2_SLICES):
        for sj, (pb, sx) in enumerate(_CT2_SLICES):
            sl = pp[(pa, pb)][:, 1 + sy:17 + sy, 1 + sx:17 + sx, :]
            acc = acc + _mm(sl.reshape(2048, 128), wct2_ref[si, sj])
    out_ref[...] = (acc + b5_ref[...][None, :]).reshape(8, 16, 16, 48)


def _vq_sc_body(dist_hbm, cb_hbm, out_hbm, dist_v, idx_v, rows_v, sem):
    wid = lax.axis_index("s") * 2 + lax.axis_index("c")

    @pl.when(wid == 0)
    def _():
        pltpu.sync_copy(dist_hbm, dist_v)
        lanes = lax.broadcasted_iota(jnp.int32, (16,), 0)
        idxs = jnp.zeros((16,), jnp.int32)
        for b in range(8):
            minv = dist_v[b, pl.ds(0, 16)]
            mini = lanes
            for c in range(1, 32):
                v = dist_v[b, pl.ds(c * 16, 16)]
                ii = lanes + c * 16
                m = v < minv
                minv = jnp.where(m, v, minv)
                mini = jnp.where(m, ii, mini)
            for shift in (8, 4, 2, 1):
                perm = jnp.bitwise_xor(lanes, shift)
                ov = _lane_shuffle(minv, perm)
                oi = _lane_shuffle(mini, perm)
                better = (ov < minv) | ((ov == minv) & (oi < mini))
                minv = jnp.where(better, ov, minv)
                mini = jnp.where(better, oi, mini)
            idxs = jnp.where(lanes == b, mini, idxs)
        idx_v[...] = idxs
        pltpu.async_copy(cb_hbm.at[idx_v], rows_v, sem).wait()
        pltpu.sync_copy(rows_v.at[pl.ds(0, 8)], out_hbm)


def _vq_lookup(dist, codebook):
    mesh = plsc.VectorSubcoreMesh(core_axis_name="c", subcore_axis_name="s")
    fn = functools.partial(
        pl.kernel,
        mesh=mesh,
        out_type=jax.ShapeDtypeStruct((8, 4096), F32),
        scratch_types=[
            pltpu.VMEM((8, 512), F32),
            pltpu.VMEM((16,), jnp.int32),
            pltpu.VMEM((16, 4096), F32),
            pltpu.SemaphoreType.DMA,
        ],
    )(_vq_sc_body)
    return fn(dist, codebook)


def _sel4(t, sel):
    z = jnp.zeros_like(t[0:1])
    return jnp.concatenate(
        [t[s:s + 1] if s is not None else z for s in sel], axis=0)


def _build_w1(enc_c1_w):
    t = enc_c1_w.transpose(2, 3, 1, 0)
    ysel = {(0, 0): (None, None, None, 0), (0, 1): (1, 2, 3, None),
            (1, 0): (None, 0, 1, 2), (1, 1): (3, None, None, None)}
    rows = []
    for py in (0, 1):
        cols = []
        for px in (0, 1):
            mats_i = []
            for i in range(2):
                ty = _sel4(t, ysel[(py, i)])
                mats_j = []
                for j in range(2):
                    txy = _sel4(ty.transpose(1, 0, 2, 3),
                                ysel[(px, j)])
                    m = txy.transpose(1, 0, 2, 3).reshape(48, 128)
                    mats_j.append(m)
                mats_i.append(jnp.stack(mats_j))
            cols.append(jnp.stack(mats_i))
        rows.append(jnp.stack(cols))
    return jnp.stack(rows)


def _build_w5(dec_ct2_w):
    u = dec_ct2_w.transpose(2, 3, 0, 1)
    mats_i = []
    for (pa, sy) in _CT2_SLICES:
        uy = _sel4(u, _CT2_ROWSEL[(pa, sy)])
        mats_j = []
        for (pb, sx) in _CT2_SLICES:
            uxy = _sel4(uy.transpose(1, 0, 2, 3),
                        _CT2_ROWSEL[(pb, sx)])
            m = uxy.transpose(2, 1, 0, 3).reshape(128, 48)
            mats_j.append(m)
        mats_i.append(jnp.stack(mats_j))
    return jnp.stack(mats_i)


def kernel(x, codebook, enc_c1_w, enc_c1_b, enc_c2_w, enc_c2_b, enc_b1a_w,
           enc_b1a_b, enc_b1b_w, enc_b1b_b, enc_b2a_w, enc_b2a_b, enc_b2b_w,
           enc_b2b_b, enc_out_w, enc_out_b, dec_oct_w, dec_oct_b, dec_b1a_w,
           dec_b1a_b, dec_b1b_w, dec_b1b_b, dec_b2a_w, dec_b2a_b, dec_b2b_w,
           dec_b2b_b, dec_ct1_w, dec_ct1_b, dec_ct2_w, dec_ct2_b):
    f32 = F32

    x_nhwc = x.transpose(0, 2, 3, 1)
    xq = x_nhwc.reshape(8, 16, 4, 16, 4, 3).transpose(0, 1, 3, 2, 4, 5)
    xq = jnp.pad(xq.reshape(8, 16, 16, 48), ((0, 0), (1, 1), (1, 1), (0, 0)))

    w1 = _build_w1(enc_c1_w)
    w2 = enc_c2_w.transpose(2, 3, 1, 0)
    ea1 = enc_b1a_w.transpose(2, 3, 1, 0)
    eb1 = enc_b1b_w[:, :, 0, 0].T
    ea2 = enc_b2a_w.transpose(2, 3, 1, 0)
    eb2 = enc_b2b_w[:, :, 0, 0].T
    wout = enc_out_w[:, :, 0, 0].T
    woct = dec_oct_w[:, :, 0, 0]
    da1 = dec_b1a_w.transpose(2, 3, 1, 0)
    db1 = dec_b1b_w[:, :, 0, 0].T
    da2 = dec_b2a_w.transpose(2, 3, 1, 0)
    db2 = dec_b2b_w[:, :, 0, 0].T
    wct1 = dec_ct1_w.transpose(2, 3, 0, 1)
    w5 = _build_w5(dec_ct2_w)
    b5 = jnp.tile(dec_ct2_b, 16)

    z_map = pl.pallas_call(
        _enc_body,
        out_shape=jax.ShapeDtypeStruct((8, 16, 16, 16), f32),
    )(xq, w1, enc_c1_b, w2, enc_c2_b, ea1, enc_b1a_b, eb1, enc_b1b_b, ea2,
      enc_b2a_b, eb2, enc_b2b_b, wout, enc_out_b)

    z_nchw = z_map.transpose(0, 3, 1, 2)
    z_flat = z_nchw.reshape(8, 4096)
    z_e = z_nchw.reshape(8, 1, 4096)

    dist = pl.pallas_call(
        _dist_body,
        out_shape=jax.ShapeDtypeStruct((8, 512), f32),
    )(z_flat, codebook)
    z_q = _vq_lookup(dist, codebook)

    latent = z_q.reshape(8, 16, 16, 16)

    o = pl.pallas_call(
        _dec_body,
        out_shape=jax.ShapeDtypeStruct((8, 16, 16, 48), f32),
    )(latent, woct, dec_oct_b, da1, dec_b1a_b, db1, dec_b1b_b, da2, dec_b2a_b,
      db2, dec_b2b_b, wct1, dec_ct1_b, w5, b5)

    o = o.reshape(8, 16, 16, 4, 4, 3).transpose(0, 1, 3, 2, 4, 5)
    r = o.reshape(8, 64, 64, 3).transpose(0, 3, 1, 2)

    return (r, z_e, z_q, z_e, z_q)
